# Initial kernel scaffold; baseline (speedup 1.0000x reference)
import jax
import jax.numpy as jnp
from jax.experimental import pallas as pl


def kernel(input_ids1, attention_mask1, input_ids2, attention_mask2, emb):
    raise NotImplementedError("write your pallas kernel here")


# speedup vs baseline: 3.6187x; 3.6187x over previous
import functools

import jax
import jax.numpy as jnp
from jax import lax
from jax.experimental import pallas as pl
from jax.experimental.pallas import tpu as pltpu
from jax.experimental.pallas import tpu_sc as plsc

_L = 200
_LP = 208
_D = 64
_NLANE = 16
_NVEC = _D // _NLANE


def _sc_body(rows_per_w, nc, ids_hbm, emb_hbm, out_hbm,
             idx_v, rows_v, out_v, sem):
    wid = lax.axis_index("s") * nc + lax.axis_index("c")
    base = wid * rows_per_w
    inv_l = jnp.float32(1.0 / _L)

    def row_body(i, carry):
        pltpu.sync_copy(ids_hbm.at[base + i], idx_v)
        h1 = pltpu.async_copy(
            emb_hbm.at[idx_v.at[pl.ds(0, _LP // 2)]],
            rows_v.at[pl.ds(0, _LP // 2)], sem)
        h2 = pltpu.async_copy(
            emb_hbm.at[idx_v.at[pl.ds(_LP // 2, _LP // 2)]],
            rows_v.at[pl.ds(_LP // 2, _LP // 2)], sem)
        h1.wait()
        h2.wait()

        def acc_body(l, accs):
            return tuple(
                accs[d] + rows_v[l, pl.ds(d * _NLANE, _NLANE)]
                for d in range(_NVEC))

        accs = lax.fori_loop(
            0, _L, acc_body,
            tuple(jnp.zeros((_NLANE,), jnp.float32) for _ in range(_NVEC)))
        for d in range(_NVEC):
            out_v[i, pl.ds(d * _NLANE, _NLANE)] = accs[d] * inv_l
        return carry

    lax.fori_loop(0, rows_per_w, row_body, 0)
    pltpu.sync_copy(out_v, out_hbm.at[pl.ds(base, rows_per_w)])


@functools.partial(jax.jit, static_argnames=())
def _run(ids, emb):
    n_rows = ids.shape[0]
    info = plsc.get_sparse_core_info()
    nc, ns = info.num_cores, info.num_subcores
    nw = nc * ns
    rows_per_w = n_rows // nw
    mesh = plsc.VectorSubcoreMesh(core_axis_name="c", subcore_axis_name="s")
    kern = functools.partial(
        pl.kernel,
        mesh=mesh,
        compiler_params=pltpu.CompilerParams(use_tc_tiling_on_sc=False),
        out_type=jax.ShapeDtypeStruct((n_rows, _D), jnp.float32),
        scratch_types=[
            pltpu.VMEM((_LP,), jnp.int32),
            pltpu.VMEM((_LP, _D), jnp.float32),
            pltpu.VMEM((rows_per_w, _D), jnp.float32),
            pltpu.SemaphoreType.DMA,
        ],
    )(functools.partial(_sc_body, rows_per_w, nc))
    return kern(ids, emb)


def kernel(input_ids1, attention_mask1, input_ids2, attention_mask2, emb):
    b = input_ids1.shape[0]
    ids = jnp.concatenate([input_ids1, input_ids2], axis=0).astype(jnp.int32)
    ids = jnp.pad(ids, ((0, 0), (0, _LP - _L)))
    out = _run(ids, emb)
    return out[:b], out[b:]
